# Initial kernel scaffold; baseline (speedup 1.0000x reference)
import jax
import jax.numpy as jnp
from jax.experimental import pallas as pl


def kernel(w1, b1, e1, w2, b2, wf1, bf1, e3, wf2, bf2, wf3, bf3, x_nchw):
    raise NotImplementedError("write your pallas kernel here")


# speedup vs baseline: 1.2956x; 1.2956x over previous
import jax
import jax.numpy as jnp
from jax.experimental import pallas as pl
from jax.experimental.pallas import tpu as pltpu

IMG_TILE = 16


def _fused_kernel(x_ref, w1c_ref, b1_ref, e1_ref, w2c_ref, b2_ref,
                  wf1c_ref, bf1_ref, e3b_ref, wf2_ref, bf2_ref,
                  wf3_ref, bf3_ref, out_ref):
    f32, bf16 = jnp.float32, jnp.bfloat16

    x = x_ref[...]
    r1 = x.shape[0]

    xb = [x.astype(bf16)] + [
        pltpu.roll(x, (r1 - i) % r1, axis=0).astype(bf16) for i in range(1, 5)]
    acc1 = jnp.dot(jnp.concatenate(xb[0:2], axis=1), w1c_ref[0:256, :],
                   preferred_element_type=f32)
    acc1 = acc1 + jnp.dot(jnp.concatenate(xb[2:4], axis=1), w1c_ref[256:512, :],
                          preferred_element_type=f32)
    acc1 = acc1 + jnp.dot(xb[4], w1c_ref[512:640, :], preferred_element_type=f32)

    yw = jnp.maximum(jnp.maximum(acc1[:, :128], acc1[:, 128:]) + b1_ref[...], 0.0)

    m1 = jnp.maximum(yw, pltpu.roll(yw, r1 - 1, axis=0)).astype(bf16)
    e1 = e1_ref[...]
    p1 = jnp.concatenate(
        [jnp.dot(e1, m1[c * 256:(c + 1) * 256, :], preferred_element_type=f32)
         for c in range(r1 // 256)], axis=0)
    r2 = p1.shape[0]

    pb = [p1.astype(bf16)] + [
        pltpu.roll(p1, (r2 - i) % r2, axis=0).astype(bf16) for i in range(1, 5)]
    acc2 = jnp.dot(jnp.concatenate(pb[0:2], axis=1), w2c_ref[0:256, :],
                   preferred_element_type=f32)
    acc2 = acc2 + jnp.dot(jnp.concatenate(pb[2:4], axis=1), w2c_ref[256:512, :],
                          preferred_element_type=f32)
    acc2 = acc2 + jnp.dot(pb[4], w2c_ref[512:640, :], preferred_element_type=f32)
    zw = jnp.maximum(jnp.maximum(acc2[:, :128], acc2[:, 128:]) + b2_ref[...], 0.0)

    m2 = jnp.maximum(zw, pltpu.roll(zw, r2 - 1, axis=0))

    fb = [m2.astype(bf16)] + [
        pltpu.roll(m2, (r2 - 2 * h) % r2, axis=0).astype(bf16) for h in range(1, 5)]
    accf = jnp.dot(jnp.concatenate(fb[0:2], axis=1), wf1c_ref[0:256, :],
                   preferred_element_type=f32)
    accf = accf + jnp.dot(jnp.concatenate(fb[2:4], axis=1), wf1c_ref[256:512, :],
                          preferred_element_type=f32)
    accf = accf + jnp.dot(fb[4], wf1c_ref[512:640, :], preferred_element_type=f32)
    z1 = jnp.maximum(accf + bf1_ref[...], 0.0)

    z1c = jnp.dot(e3b_ref[...], z1.astype(bf16), preferred_element_type=f32)
    z2 = jnp.maximum(jnp.dot(z1c.astype(bf16), wf2_ref[...],
                             preferred_element_type=f32) + bf2_ref[...], 0.0)
    z3 = jnp.dot(z2.astype(bf16), wf3_ref[...],
                 preferred_element_type=f32) + bf3_ref[...]
    out_ref[...] = z3


def kernel(w1, b1, e1, w2, b2, wf1, bf1, e3, wf2, bf2, wf3, bf3, x_nchw):
    T = IMG_TILE
    N = x_nchw.shape[0]
    n_pad = (-N) % T
    Np = N + n_pad

    x2d = jnp.transpose(x_nchw, (0, 2, 3, 1)).reshape(N, 32, 96)
    if n_pad:
        x2d = jnp.pad(x2d, ((0, n_pad), (0, 0), (0, 0)))
    xslab = jnp.pad(x2d.reshape(Np * 32, 96), ((0, 0), (0, 32)))

    w1c = jnp.zeros((5, 128, 256), jnp.bfloat16).at[:, :96, :].set(w1)
    w1c = w1c.reshape(640, 256)
    w2c = w2.reshape(640, 256)
    wf1c = wf1.reshape(640, 128)
    e3b = jnp.zeros((T, T * 14), jnp.bfloat16)
    for c in range(T // 8):
        e3b = e3b.at[c * 8:(c + 1) * 8, c * 112:(c + 1) * 112].set(e3)

    def full(*shape):
        return pl.BlockSpec(shape, lambda n, _s=len(shape): (0,) * _s)

    out = pl.pallas_call(
        _fused_kernel,
        out_shape=jax.ShapeDtypeStruct((Np, 128), jnp.float32),
        grid=(Np // T,),
        in_specs=[
            pl.BlockSpec((T * 32, 128), lambda n: (n, 0)),
            full(640, 256), full(1, 128), full(112, 256),
            full(640, 256), full(1, 128),
            full(640, 128), full(1, 128), full(T, T * 14),
            full(128, 128), full(1, 128),
            full(128, 128), full(1, 128),
        ],
        out_specs=pl.BlockSpec((T, 128), lambda n: (n, 0)),
        compiler_params=pltpu.CompilerParams(
            dimension_semantics=("parallel",)),
    )(xslab, w1c, b1, e1, w2c, b2, wf1c, bf1, e3b, wf2, bf2, wf3, bf3)
    return out[:N, :10]


# speedup vs baseline: 1.3196x; 1.0186x over previous
import numpy as np
import jax
import jax.numpy as jnp
from jax.experimental import pallas as pl
from jax.experimental.pallas import tpu as pltpu

IMG_TILE = 32
CHUNK = 8

_PERM = np.array([(j % 32) * 3 + (j // 32) for j in range(96)], dtype=np.int32)


def _chain(x, e1, w1c_ref, b1_ref, w2c_ref, b2_ref, wf1c_ref, bf1_ref, e3_ref):
    f32, bf16 = jnp.float32, jnp.bfloat16
    r1 = x.shape[0]

    def pad128(v):
        return jnp.pad(v, ((0, 0), (0, 128 - v.shape[1])))

    xb = [pad128(x.astype(bf16))] + [
        pad128(pltpu.roll(x, (r1 - i) % r1, axis=0).astype(bf16))
        for i in range(1, 5)]
    acc1 = jnp.dot(jnp.concatenate(xb[0:2], axis=1), w1c_ref[0:256, :],
                   preferred_element_type=f32)
    acc1 = acc1 + jnp.dot(jnp.concatenate(xb[2:4], axis=1), w1c_ref[256:512, :],
                          preferred_element_type=f32)
    acc1 = acc1 + jnp.dot(xb[4], w1c_ref[512:640, :], preferred_element_type=f32)

    yw = jnp.maximum(jnp.maximum(acc1[:, :128], acc1[:, 128:]) + b1_ref[...], 0.0)

    m1 = jnp.maximum(yw, pltpu.roll(yw, r1 - 1, axis=0)).astype(bf16)
    p1 = jnp.dot(e1, m1, preferred_element_type=f32)
    r2 = p1.shape[0]

    pb = [p1.astype(bf16)] + [
        pltpu.roll(p1, (r2 - i) % r2, axis=0).astype(bf16) for i in range(1, 5)]
    acc2 = jnp.dot(jnp.concatenate(pb[0:2], axis=1), w2c_ref[0:256, :],
                   preferred_element_type=f32)
    acc2 = acc2 + jnp.dot(jnp.concatenate(pb[2:4], axis=1), w2c_ref[256:512, :],
                          preferred_element_type=f32)
    acc2 = acc2 + jnp.dot(pb[4], w2c_ref[512:640, :], preferred_element_type=f32)
    zw = jnp.maximum(jnp.maximum(acc2[:, :128], acc2[:, 128:]) + b2_ref[...], 0.0)

    m2 = jnp.maximum(zw, pltpu.roll(zw, r2 - 1, axis=0))

    fb = [m2.astype(bf16)] + [
        pltpu.roll(m2, (r2 - 2 * h) % r2, axis=0).astype(bf16) for h in range(1, 5)]
    accf = jnp.dot(jnp.concatenate(fb[0:2], axis=1), wf1c_ref[0:256, :],
                   preferred_element_type=f32)
    accf = accf + jnp.dot(jnp.concatenate(fb[2:4], axis=1), wf1c_ref[256:512, :],
                          preferred_element_type=f32)
    accf = accf + jnp.dot(fb[4], wf1c_ref[512:640, :], preferred_element_type=f32)
    z1 = jnp.maximum(accf + bf1_ref[...], 0.0)

    return jnp.dot(e3_ref[...], z1.astype(bf16), preferred_element_type=f32)


def _fused_kernel(x_ref, w1c_ref, b1_ref, e1_ref, w2c_ref, b2_ref,
                  wf1c_ref, bf1_ref, e3_ref, wf2_ref, bf2_ref,
                  wf3_ref, bf3_ref, out_ref):
    f32, bf16 = jnp.float32, jnp.bfloat16
    e1 = e1_ref[...]
    z1c = jnp.concatenate(
        [_chain(x_ref[c * 256:(c + 1) * 256, :], e1, w1c_ref, b1_ref,
                w2c_ref, b2_ref, wf1c_ref, bf1_ref, e3_ref)
         for c in range(IMG_TILE // CHUNK)], axis=0)
    z2 = jnp.maximum(jnp.dot(z1c.astype(bf16), wf2_ref[...],
                             preferred_element_type=f32) + bf2_ref[...], 0.0)
    z3 = jnp.dot(z2.astype(bf16), wf3_ref[...],
                 preferred_element_type=f32) + bf3_ref[...]
    out_ref[...] = z3


def kernel(w1, b1, e1, w2, b2, wf1, bf1, e3, wf2, bf2, wf3, bf3, x_nchw):
    T = IMG_TILE
    N = x_nchw.shape[0]
    n_pad = (-N) % T
    Np = N + n_pad

    x2d = jnp.transpose(x_nchw, (0, 2, 1, 3)).reshape(N, 32, 96)
    if n_pad:
        x2d = jnp.pad(x2d, ((0, n_pad), (0, 0), (0, 0)))
    xslab = x2d.reshape(Np * 32, 96)

    w1c = jnp.zeros((5, 128, 256), jnp.bfloat16).at[:, :96, :].set(w1[:, _PERM, :])
    w1c = w1c.reshape(640, 256)
    w2c = w2.reshape(640, 256)
    wf1c = wf1.reshape(640, 128)

    def full(*shape):
        return pl.BlockSpec(shape, lambda n, _s=len(shape): (0,) * _s)

    out = pl.pallas_call(
        _fused_kernel,
        out_shape=jax.ShapeDtypeStruct((Np, 128), jnp.float32),
        grid=(Np // T,),
        in_specs=[
            pl.BlockSpec((T * 32, 96), lambda n: (n, 0)),
            full(640, 256), full(1, 128), full(112, 256),
            full(640, 256), full(1, 128),
            full(640, 128), full(1, 128), full(8, 112),
            full(128, 128), full(1, 128),
            full(128, 128), full(1, 128),
        ],
        out_specs=pl.BlockSpec((T, 128), lambda n: (n, 0)),
        compiler_params=pltpu.CompilerParams(
            dimension_semantics=("parallel",)),
    )(xslab, w1c, b1, e1, w2c, b2, wf1c, bf1, e3, wf2, bf2, wf3, bf3)
    return out[:N, :10]


# speedup vs baseline: 1.3730x; 1.0404x over previous
import numpy as np
import jax
import jax.numpy as jnp
from jax.experimental import pallas as pl
from jax.experimental.pallas import tpu as pltpu

IMG_TILE = 32
CHUNK = 16

_PERM = np.array([(j % 32) * 3 + (j // 32) for j in range(96)], dtype=np.int32)


def _chain(x, e1, w1c_ref, b1_ref, w2c_ref, b2_ref, wf1c_ref, bf1_ref,
           e3b, wf2_ref, bf2_ref, wf3_ref, bf3_ref):
    f32, bf16 = jnp.float32, jnp.bfloat16
    r1 = x.shape[0]

    def pad128(v):
        return jnp.pad(v, ((0, 0), (0, 128 - v.shape[1])))

    xb = [pad128(x.astype(bf16))] + [
        pad128(pltpu.roll(x, (r1 - i) % r1, axis=0).astype(bf16))
        for i in range(1, 5)]
    acc1 = jnp.dot(jnp.concatenate(xb[0:2], axis=1), w1c_ref[0:256, :],
                   preferred_element_type=f32)
    acc1 = acc1 + jnp.dot(jnp.concatenate(xb[2:4], axis=1), w1c_ref[256:512, :],
                          preferred_element_type=f32)
    acc1 = acc1 + jnp.dot(xb[4], w1c_ref[512:640, :], preferred_element_type=f32)

    yw = jnp.maximum(jnp.maximum(acc1[:, :128], acc1[:, 128:]) + b1_ref[...], 0.0)

    m1 = jnp.maximum(yw, pltpu.roll(yw, r1 - 1, axis=0)).astype(bf16)
    p1 = jnp.concatenate(
        [jnp.dot(e1, m1[c * 256:(c + 1) * 256, :], preferred_element_type=f32)
         for c in range(r1 // 256)], axis=0)
    r2 = p1.shape[0]

    pb = [p1.astype(bf16)] + [
        pltpu.roll(p1, (r2 - i) % r2, axis=0).astype(bf16) for i in range(1, 5)]
    acc2 = jnp.dot(jnp.concatenate(pb[0:2], axis=1), w2c_ref[0:256, :],
                   preferred_element_type=f32)
    acc2 = acc2 + jnp.dot(jnp.concatenate(pb[2:4], axis=1), w2c_ref[256:512, :],
                          preferred_element_type=f32)
    acc2 = acc2 + jnp.dot(pb[4], w2c_ref[512:640, :], preferred_element_type=f32)
    zw = jnp.maximum(jnp.maximum(acc2[:, :128], acc2[:, 128:]) + b2_ref[...], 0.0)

    m2 = jnp.maximum(zw, pltpu.roll(zw, r2 - 1, axis=0))

    fb = [m2.astype(bf16)] + [
        pltpu.roll(m2, (r2 - 2 * h) % r2, axis=0).astype(bf16) for h in range(1, 5)]
    accf = jnp.dot(jnp.concatenate(fb[0:2], axis=1), wf1c_ref[0:256, :],
                   preferred_element_type=f32)
    accf = accf + jnp.dot(jnp.concatenate(fb[2:4], axis=1), wf1c_ref[256:512, :],
                          preferred_element_type=f32)
    accf = accf + jnp.dot(fb[4], wf1c_ref[512:640, :], preferred_element_type=f32)
    z1 = jnp.maximum(accf + bf1_ref[...], 0.0)

    z1c = jnp.dot(e3b, z1.astype(bf16), preferred_element_type=f32)
    z2 = jnp.maximum(jnp.dot(z1c.astype(bf16), wf2_ref[...],
                             preferred_element_type=f32) + bf2_ref[...], 0.0)
    return jnp.dot(z2.astype(bf16), wf3_ref[...],
                   preferred_element_type=f32) + bf3_ref[...]


def _fused_kernel(x_ref, w1c_ref, b1_ref, e1_ref, w2c_ref, b2_ref,
                  wf1c_ref, bf1_ref, e3b_ref, wf2_ref, bf2_ref,
                  wf3_ref, bf3_ref, out_ref):
    e1 = e1_ref[...]
    e3b = e3b_ref[...]
    for c in range(IMG_TILE // CHUNK):
        out_ref[c * CHUNK:(c + 1) * CHUNK, :] = _chain(
            x_ref[c * CHUNK * 32:(c + 1) * CHUNK * 32, :], e1, w1c_ref, b1_ref,
            w2c_ref, b2_ref, wf1c_ref, bf1_ref, e3b, wf2_ref, bf2_ref,
            wf3_ref, bf3_ref)


def kernel(w1, b1, e1, w2, b2, wf1, bf1, e3, wf2, bf2, wf3, bf3, x_nchw):
    T = IMG_TILE
    N = x_nchw.shape[0]
    n_pad = (-N) % T
    Np = N + n_pad

    x2d = jnp.transpose(x_nchw, (0, 2, 1, 3)).reshape(N, 32, 96)
    if n_pad:
        x2d = jnp.pad(x2d, ((0, n_pad), (0, 0), (0, 0)))
    xslab = x2d.reshape(Np * 32, 96)

    w1c = jnp.zeros((5, 128, 256), jnp.bfloat16).at[:, :96, :].set(w1[:, _PERM, :])
    w1c = w1c.reshape(640, 256)
    w2c = w2.reshape(640, 256)
    wf1c = wf1.reshape(640, 128)
    e3b = jnp.zeros((CHUNK, CHUNK * 14), jnp.bfloat16)
    for c in range(CHUNK // 8):
        e3b = e3b.at[c * 8:(c + 1) * 8, c * 112:(c + 1) * 112].set(e3)

    def full(*shape):
        return pl.BlockSpec(shape, lambda n, _s=len(shape): (0,) * _s)

    out = pl.pallas_call(
        _fused_kernel,
        out_shape=jax.ShapeDtypeStruct((Np, 128), jnp.float32),
        grid=(Np // T,),
        in_specs=[
            pl.BlockSpec((T * 32, 96), lambda n: (n, 0)),
            full(640, 256), full(1, 128), full(112, 256),
            full(640, 256), full(1, 128),
            full(640, 128), full(1, 128), full(CHUNK, CHUNK * 14),
            full(128, 128), full(1, 128),
            full(128, 128), full(1, 128),
        ],
        out_specs=pl.BlockSpec((T, 128), lambda n: (n, 0)),
        compiler_params=pltpu.CompilerParams(
            dimension_semantics=("parallel",)),
    )(xslab, w1c, b1, e1, w2c, b2, wf1c, bf1, e3b, wf2, bf2, wf3, bf3)
    return out[:N, :10]


# speedup vs baseline: 1.4059x; 1.0240x over previous
import numpy as np
import jax
import jax.numpy as jnp
from jax.experimental import pallas as pl
from jax.experimental.pallas import tpu as pltpu

IMG_TILE = 64
CHUNK = 16

_PERM = np.array([(j % 32) * 3 + (j // 32) for j in range(96)], dtype=np.int32)

_E1 = np.zeros((112, 256), np.float32)
for _i in range(8):
    for _r in range(14):
        _E1[_i * 14 + _r, _i * 32 + 2 * _r] = 1.0
_E3B = np.zeros((CHUNK, CHUNK * 14), np.float32)
for _i in range(CHUNK):
    _E3B[_i, _i * 14] = 1.0


def _chain(x, e1, w1c_ref, b1_ref, w2c_ref, b2_ref, wf1c_ref, bf1_ref,
           e3b, wf2_ref, bf2_ref, wf3_ref, bf3_ref):
    f32, bf16 = jnp.float32, jnp.bfloat16
    r1 = x.shape[0]

    def pad128(v):
        return jnp.pad(v, ((0, 0), (0, 128 - v.shape[1])))

    xb = [pad128(x.astype(bf16))] + [
        pad128(pltpu.roll(x, (r1 - i) % r1, axis=0).astype(bf16))
        for i in range(1, 5)]
    acc1 = jnp.dot(jnp.concatenate(xb[0:2], axis=1), w1c_ref[0:256, :],
                   preferred_element_type=f32)
    acc1 = acc1 + jnp.dot(jnp.concatenate(xb[2:4], axis=1), w1c_ref[256:512, :],
                          preferred_element_type=f32)
    acc1 = acc1 + jnp.dot(xb[4], w1c_ref[512:640, :], preferred_element_type=f32)

    yw = jnp.maximum(jnp.maximum(acc1[:, :128], acc1[:, 128:]) + b1_ref[...], 0.0)

    m1 = jnp.maximum(yw, pltpu.roll(yw, r1 - 1, axis=0)).astype(bf16)
    p1 = jnp.concatenate(
        [jnp.dot(e1, m1[c * 256:(c + 1) * 256, :], preferred_element_type=f32)
         for c in range(r1 // 256)], axis=0)
    r2 = p1.shape[0]

    pb = [p1.astype(bf16)] + [
        pltpu.roll(p1, (r2 - i) % r2, axis=0).astype(bf16) for i in range(1, 5)]
    acc2 = jnp.dot(jnp.concatenate(pb[0:2], axis=1), w2c_ref[0:256, :],
                   preferred_element_type=f32)
    acc2 = acc2 + jnp.dot(jnp.concatenate(pb[2:4], axis=1), w2c_ref[256:512, :],
                          preferred_element_type=f32)
    acc2 = acc2 + jnp.dot(pb[4], w2c_ref[512:640, :], preferred_element_type=f32)
    zw = jnp.maximum(jnp.maximum(acc2[:, :128], acc2[:, 128:]) + b2_ref[...], 0.0)

    m2 = jnp.maximum(zw, pltpu.roll(zw, r2 - 1, axis=0))

    fb = [m2.astype(bf16)] + [
        pltpu.roll(m2, (r2 - 2 * h) % r2, axis=0).astype(bf16) for h in range(1, 5)]
    accf = jnp.dot(jnp.concatenate(fb[0:2], axis=1), wf1c_ref[0:256, :],
                   preferred_element_type=f32)
    accf = accf + jnp.dot(jnp.concatenate(fb[2:4], axis=1), wf1c_ref[256:512, :],
                          preferred_element_type=f32)
    accf = accf + jnp.dot(fb[4], wf1c_ref[512:640, :], preferred_element_type=f32)
    z1 = jnp.maximum(accf + bf1_ref[...], 0.0)

    z1c = jnp.dot(e3b, z1.astype(bf16), preferred_element_type=f32)
    z2 = jnp.maximum(jnp.dot(z1c.astype(bf16), wf2_ref[...],
                             preferred_element_type=f32) + bf2_ref[...], 0.0)
    return jnp.dot(z2.astype(bf16), wf3_ref[...],
                   preferred_element_type=f32) + bf3_ref[...]


def _fused_kernel(x_ref, w1c_ref, b1_ref, e1_ref, w2c_ref, b2_ref,
                  wf1c_ref, bf1_ref, e3b_ref, wf2_ref, bf2_ref,
                  wf3_ref, bf3_ref, out_ref):
    e1 = e1_ref[...]
    e3b = e3b_ref[...]
    for c in range(IMG_TILE // CHUNK):
        out_ref[c * CHUNK:(c + 1) * CHUNK, :] = _chain(
            x_ref[c * CHUNK * 32:(c + 1) * CHUNK * 32, :], e1, w1c_ref, b1_ref,
            w2c_ref, b2_ref, wf1c_ref, bf1_ref, e3b, wf2_ref, bf2_ref,
            wf3_ref, bf3_ref)


def kernel(w1, b1, e1, w2, b2, wf1, bf1, e3, wf2, bf2, wf3, bf3, x_nchw):
    T = IMG_TILE
    N = x_nchw.shape[0]
    n_pad = (-N) % T
    Np = N + n_pad

    x2d = jnp.transpose(x_nchw, (0, 2, 1, 3)).reshape(N, 32, 96)
    if n_pad:
        x2d = jnp.pad(x2d, ((0, n_pad), (0, 0), (0, 0)))
    xslab = x2d.reshape(Np * 32, 96)

    w1c = jnp.zeros((5, 128, 256), jnp.bfloat16).at[:, :96, :].set(w1[:, _PERM, :])
    w1c = w1c.reshape(640, 256)
    w2c = w2.reshape(640, 256)
    wf1c = wf1.reshape(640, 128)
    e1c = jnp.asarray(_E1, jnp.bfloat16)
    e3b = jnp.asarray(_E3B, jnp.bfloat16)

    def full(*shape):
        return pl.BlockSpec(shape, lambda n, _s=len(shape): (0,) * _s)

    out = pl.pallas_call(
        _fused_kernel,
        out_shape=jax.ShapeDtypeStruct((Np, 128), jnp.float32),
        grid=(Np // T,),
        in_specs=[
            pl.BlockSpec((T * 32, 96), lambda n: (n, 0)),
            full(640, 256), full(1, 128), full(112, 256),
            full(640, 256), full(1, 128),
            full(640, 128), full(1, 128), full(CHUNK, CHUNK * 14),
            full(128, 128), full(1, 128),
            full(128, 128), full(1, 128),
        ],
        out_specs=pl.BlockSpec((T, 128), lambda n: (n, 0)),
        compiler_params=pltpu.CompilerParams(
            dimension_semantics=("parallel",)),
    )(xslab, w1c, b1, e1c, w2c, b2, wf1c, bf1, e3b, wf2, bf2, wf3, bf3)
    return out[:N, :10]


# speedup vs baseline: 1.9846x; 1.4116x over previous
import numpy as np
import jax
import jax.numpy as jnp
from jax.experimental import pallas as pl
from jax.experimental.pallas import tpu as pltpu

IMG_TILE = 64
CHUNK = 16

_PERM = np.array([(j % 32) * 3 + (j // 32) for j in range(96)], dtype=np.int32)

_E1 = np.zeros((112, 256), np.float32)
for _i in range(8):
    for _r in range(14):
        _E1[_i * 14 + _r, _i * 32 + 2 * _r] = 1.0
_E3B = np.zeros((CHUNK, CHUNK * 14), np.float32)
for _i in range(CHUNK):
    _E3B[_i, _i * 14] = 1.0


def _chain(x, e1, w1c_ref, b1_ref, w2c_ref, b2_ref, wf1c_ref, bf1_ref,
           e3b, wf2_ref, bf2_ref, wf3_ref, bf3_ref):
    f32, bf16 = jnp.float32, jnp.bfloat16
    r1 = x.shape[0]

    def pad128(v):
        return jnp.pad(v, ((0, 0), (0, 128 - v.shape[1])))

    xb = [pad128(x.astype(bf16))] + [
        pad128(pltpu.roll(x, (r1 - i) % r1, axis=0).astype(bf16))
        for i in range(1, 5)]
    acc1 = jnp.dot(jnp.concatenate(xb[0:2], axis=1), w1c_ref[0:256, :],
                   preferred_element_type=f32)
    acc1 = acc1 + jnp.dot(jnp.concatenate(xb[2:4], axis=1), w1c_ref[256:512, :],
                          preferred_element_type=f32)
    acc1 = acc1 + jnp.dot(xb[4], w1c_ref[512:640, :], preferred_element_type=f32)

    yw = jnp.maximum(jnp.maximum(acc1[:, :128], acc1[:, 128:]) + b1_ref[...], 0.0)

    m1 = jnp.maximum(yw, pltpu.roll(yw, r1 - 1, axis=0)).astype(bf16)
    p1 = jnp.concatenate(
        [jnp.dot(e1, m1[c * 256:(c + 1) * 256, :], preferred_element_type=f32)
         for c in range(r1 // 256)], axis=0)
    r2 = p1.shape[0]

    pb = [p1.astype(bf16)] + [
        pltpu.roll(p1, (r2 - i) % r2, axis=0).astype(bf16) for i in range(1, 5)]
    acc2 = jnp.dot(jnp.concatenate(pb[0:2], axis=1), w2c_ref[0:256, :],
                   preferred_element_type=f32)
    acc2 = acc2 + jnp.dot(jnp.concatenate(pb[2:4], axis=1), w2c_ref[256:512, :],
                          preferred_element_type=f32)
    acc2 = acc2 + jnp.dot(pb[4], w2c_ref[512:640, :], preferred_element_type=f32)
    zw = jnp.maximum(jnp.maximum(acc2[:, :128], acc2[:, 128:]) + b2_ref[...], 0.0)

    m2 = jnp.maximum(zw, pltpu.roll(zw, r2 - 1, axis=0))

    fb = [m2.astype(bf16)] + [
        pltpu.roll(m2, (r2 - 2 * h) % r2, axis=0).astype(bf16) for h in range(1, 5)]
    accf = jnp.dot(jnp.concatenate(fb[0:2], axis=1), wf1c_ref[0:256, :],
                   preferred_element_type=f32)
    accf = accf + jnp.dot(jnp.concatenate(fb[2:4], axis=1), wf1c_ref[256:512, :],
                          preferred_element_type=f32)
    accf = accf + jnp.dot(fb[4], wf1c_ref[512:640, :], preferred_element_type=f32)
    z1 = jnp.maximum(accf + bf1_ref[...], 0.0)

    z1c = jnp.dot(e3b, z1.astype(bf16), preferred_element_type=f32)
    z2 = jnp.maximum(jnp.dot(z1c.astype(bf16), wf2_ref[...],
                             preferred_element_type=f32) + bf2_ref[...], 0.0)
    return jnp.dot(z2.astype(bf16), wf3_ref[...],
                   preferred_element_type=f32) + bf3_ref[...]


def _fused_kernel(x_ref, w1c_ref, b1_ref, e1_ref, w2c_ref, b2_ref,
                  wf1c_ref, bf1_ref, e3b_ref, wf2_ref, bf2_ref,
                  wf3_ref, bf3_ref, out_ref):
    e1 = e1_ref[...]
    e3b = e3b_ref[...]
    for c in range(IMG_TILE // CHUNK):
        xc = x_ref[c * CHUNK:(c + 1) * CHUNK].reshape(CHUNK * 32, 96)
        out_ref[c * CHUNK:(c + 1) * CHUNK, :] = _chain(
            xc, e1, w1c_ref, b1_ref,
            w2c_ref, b2_ref, wf1c_ref, bf1_ref, e3b, wf2_ref, bf2_ref,
            wf3_ref, bf3_ref)


def kernel(w1, b1, e1, w2, b2, wf1, bf1, e3, wf2, bf2, wf3, bf3, x_nchw):
    T = IMG_TILE
    N = x_nchw.shape[0]
    n_pad = (-N) % T
    Np = N + n_pad

    x3d = jnp.transpose(x_nchw, (0, 2, 1, 3)).reshape(N, 32, 96)
    if n_pad:
        x3d = jnp.pad(x3d, ((0, n_pad), (0, 0), (0, 0)))

    w1c = jnp.zeros((5, 128, 256), jnp.bfloat16).at[:, :96, :].set(w1[:, _PERM, :])
    w1c = w1c.reshape(640, 256)
    w2c = w2.reshape(640, 256)
    wf1c = wf1.reshape(640, 128)
    e1c = jnp.asarray(_E1, jnp.bfloat16)
    e3b = jnp.asarray(_E3B, jnp.bfloat16)

    def full(*shape):
        return pl.BlockSpec(shape, lambda n, _s=len(shape): (0,) * _s)

    out = pl.pallas_call(
        _fused_kernel,
        out_shape=jax.ShapeDtypeStruct((Np, 128), jnp.float32),
        grid=(Np // T,),
        in_specs=[
            pl.BlockSpec((T, 32, 96), lambda n: (n, 0, 0)),
            full(640, 256), full(1, 128), full(112, 256),
            full(640, 256), full(1, 128),
            full(640, 128), full(1, 128), full(CHUNK, CHUNK * 14),
            full(128, 128), full(1, 128),
            full(128, 128), full(1, 128),
        ],
        out_specs=pl.BlockSpec((T, 128), lambda n: (n, 0)),
        compiler_params=pltpu.CompilerParams(
            dimension_semantics=("parallel",)),
    )(x3d, w1c, b1, e1c, w2c, b2, wf1c, bf1, e3b, wf2, bf2, wf3, bf3)
    return out[:N, :10]
